# baseline (speedup 1.0000x reference)
import jax
import jax.numpy as jnp
from jax.experimental import pallas as pl
from jax.experimental.pallas import tpu as pltpu

TOKEN_BLOCK = 512


def _router_kernel(x_ref, w_ref, noise_ref, out_ref):
    i = pl.program_id(0)
    logits_t = jax.lax.dot_general(
        w_ref[...],
        x_ref[...],
        dimension_numbers=(((1,), (1,)), ((), ())),
        preferred_element_type=jnp.float32,
    )
    cols = pl.ds(i * TOKEN_BLOCK, TOKEN_BLOCK)
    out_ref[...] = logits_t + noise_ref[:, cols]


def kernel(x, W, noise):
    tokens, d_model = x.shape
    n_experts = W.shape[0]
    noise_t = noise.T
    grid = (tokens // TOKEN_BLOCK,)
    out_t = pl.pallas_call(
        _router_kernel,
        grid=grid,
        in_specs=[
            pl.BlockSpec((TOKEN_BLOCK, d_model), lambda i: (i, 0)),
            pl.BlockSpec((n_experts, d_model), lambda i: (0, 0)),
            pl.BlockSpec((n_experts, tokens), lambda i: (0, 0)),
        ],
        out_specs=pl.BlockSpec((n_experts, TOKEN_BLOCK), lambda i: (0, i)),
        out_shape=jax.ShapeDtypeStruct((n_experts, tokens), jnp.float32),
        compiler_params=pltpu.CompilerParams(
            dimension_semantics=("arbitrary",),
        ),
    )(x, W, noise_t)
    return out_t.T


# speedup vs baseline: 1.1834x; 1.1834x over previous
import jax
import jax.numpy as jnp
from jax.experimental import pallas as pl
from jax.experimental.pallas import tpu as pltpu

TOKEN_BLOCK = 1024


HALF = TOKEN_BLOCK // 2


def _router_kernel(x0_ref, x1_ref, w_ref, noise_ref, out_ref):
    i = pl.program_id(0)
    w = w_ref[...]
    for j, x_ref in enumerate((x0_ref, x1_ref)):
        logits_t = jax.lax.dot_general(
            w,
            x_ref[...],
            dimension_numbers=(((1,), (1,)), ((), ())),
            preferred_element_type=jnp.float32,
        )
        cols = pl.ds(i * TOKEN_BLOCK + j * HALF, HALF)
        out_ref[:, pl.ds(j * HALF, HALF)] = logits_t + noise_ref[:, cols]


def kernel(x, W, noise):
    tokens, d_model = x.shape
    n_experts = W.shape[0]
    noise_t = noise.T
    grid = (tokens // TOKEN_BLOCK,)
    out_t = pl.pallas_call(
        _router_kernel,
        grid=grid,
        in_specs=[
            pl.BlockSpec((TOKEN_BLOCK // 2, d_model), lambda i: (2 * i, 0)),
            pl.BlockSpec((TOKEN_BLOCK // 2, d_model), lambda i: (2 * i + 1, 0)),
            pl.BlockSpec((n_experts, d_model), lambda i: (0, 0)),
            pl.BlockSpec((n_experts, tokens), lambda i: (0, 0)),
        ],
        out_specs=pl.BlockSpec((n_experts, TOKEN_BLOCK), lambda i: (0, i)),
        out_shape=jax.ShapeDtypeStruct((n_experts, tokens), jnp.float32),
        compiler_params=pltpu.CompilerParams(
            dimension_semantics=("arbitrary",),
        ),
    )(x, x, W, noise_t)
    return out_t.T
